# baseline (speedup 1.0000x reference)
import functools

import jax
import jax.numpy as jnp
from jax import lax
from jax.experimental import pallas as pl
from jax.experimental.pallas import tpu as pltpu
from jax.experimental.pallas import tpu_sc as plsc

NUM_EMB = 1000000
DIM = 32
NC = 2
NS = 16
NW = NC * NS
UNIT = 1024
TT = 25
BT = 128
N_UNITS = TT * BT
PER_W = N_UNITS // NW


def _make_lookup():
  mesh = plsc.VectorSubcoreMesh(core_axis_name="c", subcore_axis_name="s")

  @functools.partial(
      pl.kernel,
      mesh=mesh,
      out_type=jax.ShapeDtypeStruct((200, 4, BT, 8, 128), jnp.float32),
      compiler_params=pltpu.CompilerParams(
          use_tc_tiling_on_sc=False, needs_layout_passes=False),
      scratch_types=[
          pltpu.VMEM((2, UNIT), jnp.int32),
          pltpu.VMEM((2, UNIT, DIM), jnp.float32),
          pltpu.VMEM((8, 4, 8, 128), jnp.float32),
          [pltpu.SemaphoreType.DMA] * 2,
          [pltpu.SemaphoreType.DMA] * 2,
          pltpu.SemaphoreType.DMA,
      ],
  )
  def lookup(idx_hbm, table_hbm, z_hbm, idx_v, rows_v, zbuf, sidx, sgat, szout):
    wid = lax.axis_index("s") * NC + lax.axis_index("c")
    base = wid * PER_W

    def idx_copy(u, j):
      return pltpu.make_async_copy(
          idx_hbm.at[pl.ds(u * UNIT, UNIT)], idx_v.at[j], sidx[j])

    def gather_copy(j):
      return pltpu.make_async_copy(
          table_hbm.at[idx_v.at[j]], rows_v.at[j], sgat[j])

    def zout_copy(u):
      tt = u // BT
      bt = u % BT
      return pltpu.make_async_copy(
          zbuf, z_hbm.at[pl.ds(tt * 8, 8), :, bt], szout)

    lanes = lax.iota(jnp.int32, 16)

    def transpose_unit(j):
      def body(sd, carry):
        s = sd // DIM
        d = sd % DIM
        dt = d // 8
        s2 = d % 8
        row0 = s * 128
        dcol = jnp.full((16,), d, jnp.int32)
        for v in range(8):
          ridx = row0 + v * 16 + lanes
          vals = plsc.load_gather(rows_v.at[j], [ridx, dcol])
          zbuf[s, dt, s2, pl.ds(v * 16, 16)] = vals
        return carry

      lax.fori_loop(0, 8 * DIM, body, 0, unroll=False)

    idx_copy(base, 0).start()
    idx_copy(base + 1, 1).start()
    idx_copy(base, 0).wait()
    gather_copy(0).start()

    def step(i, j):
      u = base + i
      j2 = 1 - j
      gather_copy(j).wait()
      def _next_gather():
        idx_copy(u + 1, j2).wait()
        gather_copy(j2).start()

      pl.when(i + 1 < PER_W)(_next_gather)
      pl.when(i + 2 < PER_W)(lambda: idx_copy(u + 2, j).start())
      pl.when(i > 0)(lambda: zout_copy(u - 1).wait())
      transpose_unit(j)
      zout_copy(u).start()

    def pair(g, carry):
      step(g * 2, 0)
      step(g * 2 + 1, 1)
      return carry

    lax.fori_loop(0, PER_W // 2, pair, 0, unroll=False)
    zout_copy(base + PER_W - 1).wait()

  return lookup


def kernel(token_ids, weight):
  tid_lin = (
      token_ids.T.reshape(TT, 8, BT, 128).transpose(0, 2, 1, 3).reshape(-1)
  ).astype(jnp.int32)
  z = _make_lookup()(tid_lin, weight)
  return z.transpose(2, 4, 0, 1, 3).reshape(16384, 200, DIM)


# speedup vs baseline: 1.0411x; 1.0411x over previous
import functools

import jax
import jax.numpy as jnp
from jax import lax
from jax.experimental import pallas as pl
from jax.experimental.pallas import tpu as pltpu
from jax.experimental.pallas import tpu_sc as plsc

NUM_EMB = 1000000
DIM = 32
NC = 2
NS = 16
NW = NC * NS
UNIT = 1024
TT = 25
BT = 128
N_UNITS = TT * BT
PER_W = N_UNITS // NW


def _make_lookup():
  mesh = plsc.VectorSubcoreMesh(core_axis_name="c", subcore_axis_name="s")

  @functools.partial(
      pl.kernel,
      mesh=mesh,
      out_type=jax.ShapeDtypeStruct((200, 4, BT, 8, 128), jnp.float32),
      compiler_params=pltpu.CompilerParams(
          use_tc_tiling_on_sc=False, needs_layout_passes=False),
      scratch_types=[
          pltpu.VMEM((2, UNIT), jnp.int32),
          pltpu.VMEM((2, UNIT, DIM), jnp.float32),
          pltpu.VMEM((2, 4, 4, 8, 128), jnp.float32),
          [pltpu.SemaphoreType.DMA] * 2,
          [pltpu.SemaphoreType.DMA] * 2,
          [pltpu.SemaphoreType.DMA] * 2,
      ],
  )
  def lookup(idx_hbm, table_hbm, z_hbm, idx_v, rows_v, zbuf, sidx, sgat, szout):
    wid = lax.axis_index("s") * NC + lax.axis_index("c")
    base = wid * PER_W

    def idx_copy(u, j):
      return pltpu.make_async_copy(
          idx_hbm.at[pl.ds(u * UNIT, UNIT)], idx_v.at[j], sidx[j])

    def gather_copy(j):
      return pltpu.make_async_copy(
          table_hbm.at[idx_v.at[j]], rows_v.at[j], sgat[j])

    def zout_copy(u, h):
      tt = u // BT
      bt = u % BT
      return pltpu.make_async_copy(
          zbuf.at[h], z_hbm.at[pl.ds(tt * 8 + h * 4, 4), :, bt], szout[h])

    lanes = lax.iota(jnp.int32, 16)
    dcols = [jnp.full((16,), d, jnp.int32) for d in range(DIM)]

    def transpose_half(j, h):
      def body(sv, carry):
        si = sv // 8
        v = sv % 8
        ridx = (h * 4 + si) * 128 + v * 16 + lanes
        for d in range(DIM):
          vals = plsc.load_gather(rows_v.at[j], [ridx, dcols[d]])
          zbuf[h, si, d // 8, d % 8, pl.ds(v * 16, 16)] = vals
        return carry

      lax.fori_loop(0, 32, body, 0, unroll=False)

    idx_copy(base, 0).start()
    idx_copy(base + 1, 1).start()
    idx_copy(base, 0).wait()
    gather_copy(0).start()

    def step(i, j):
      u = base + i
      j2 = 1 - j
      gather_copy(j).wait()
      def _next_gather():
        idx_copy(u + 1, j2).wait()
        gather_copy(j2).start()

      pl.when(i + 1 < PER_W)(_next_gather)
      pl.when(i + 2 < PER_W)(lambda: idx_copy(u + 2, j).start())
      for h in range(2):
        pl.when(i > 0)(lambda h=h: zout_copy(u - 1, h).wait())
        transpose_half(j, h)
        zout_copy(u, h).start()

    def pair(g, carry):
      step(g * 2, 0)
      step(g * 2 + 1, 1)
      return carry

    lax.fori_loop(0, PER_W // 2, pair, 0, unroll=False)
    zout_copy(base + PER_W - 1, 0).wait()
    zout_copy(base + PER_W - 1, 1).wait()

  return lookup


def kernel(token_ids, weight):
  tid_lin = (
      token_ids.T.reshape(TT, 8, BT, 128).transpose(0, 2, 1, 3).reshape(-1)
  ).astype(jnp.int32)
  z = _make_lookup()(tid_lin, weight)
  return z.transpose(2, 4, 0, 1, 3).reshape(16384, 200, DIM)


# speedup vs baseline: 3.0801x; 2.9585x over previous
import functools

import jax
import jax.numpy as jnp
from jax import lax
from jax.experimental import pallas as pl
from jax.experimental.pallas import tpu as pltpu
from jax.experimental.pallas import tpu_sc as plsc

NUM_EMB = 1000000
DIM = 32
NC = 2
NS = 16
NW = NC * NS
UNIT = 1024
TT = 25
BT = 128
N_UNITS = TT * BT
PER_W = N_UNITS // NW


def _make_lookup():
  mesh = plsc.VectorSubcoreMesh(core_axis_name="c", subcore_axis_name="s")

  @functools.partial(
      pl.kernel,
      mesh=mesh,
      out_type=jax.ShapeDtypeStruct((200, 4, BT, 8, 128), jnp.float32),
      compiler_params=pltpu.CompilerParams(
          use_tc_tiling_on_sc=False, needs_layout_passes=False),
      scratch_types=[
          pltpu.VMEM((2, UNIT), jnp.int32),
          pltpu.VMEM((2, UNIT, DIM), jnp.float32),
          pltpu.VMEM((2, 4, 4, 8, 129), jnp.float32),
          [pltpu.SemaphoreType.DMA] * 2,
          [pltpu.SemaphoreType.DMA] * 2,
          [pltpu.SemaphoreType.DMA] * 2,
      ],
  )
  def lookup(idx_hbm, table_hbm, z_hbm, idx_v, rows_v, zbuf, sidx, sgat, szout):
    wid = lax.axis_index("s") * NC + lax.axis_index("c")
    base = wid * PER_W

    def idx_copy(u, j):
      return pltpu.make_async_copy(
          idx_hbm.at[pl.ds(u * UNIT, UNIT)], idx_v.at[j], sidx[j])

    def gather_copy(j):
      return pltpu.make_async_copy(
          table_hbm.at[idx_v.at[j]], rows_v.at[j], sgat[j])

    def zout_copy(u, h):
      tt = u // BT
      bt = u % BT
      return pltpu.make_async_copy(
          zbuf.at[h, :, :, :, pl.ds(0, 128)],
          z_hbm.at[pl.ds(tt * 8 + h * 4, 4), :, bt], szout[h])

    dlo = lax.iota(jnp.int32, 16)
    dhi = dlo + 16

    def transpose_half(j, h):
      def body(tv, carry):
        si = tv // 128
        l = tv % 128
        tok = h * 512 + tv
        lv = jnp.full((16,), l, jnp.int32)
        a = rows_v[j, tok, pl.ds(0, 16)]
        b = rows_v[j, tok, pl.ds(16, 16)]
        zsub = zbuf.at[h, si]
        plsc.store_scatter(zsub, [dlo // 8, dlo % 8, lv], a)
        plsc.store_scatter(zsub, [dhi // 8, dhi % 8, lv], b)
        return carry

      lax.fori_loop(0, 512, body, 0, unroll=False)

    idx_copy(base, 0).start()
    idx_copy(base + 1, 1).start()
    idx_copy(base, 0).wait()
    gather_copy(0).start()

    def step(i, j):
      u = base + i
      j2 = 1 - j
      gather_copy(j).wait()
      def _next_gather():
        idx_copy(u + 1, j2).wait()
        gather_copy(j2).start()

      pl.when(i + 1 < PER_W)(_next_gather)
      pl.when(i + 2 < PER_W)(lambda: idx_copy(u + 2, j).start())
      for h in range(2):
        pl.when(i > 0)(lambda h=h: zout_copy(u - 1, h).wait())
        transpose_half(j, h)
        zout_copy(u, h).start()

    def pair(g, carry):
      step(g * 2, 0)
      step(g * 2 + 1, 1)
      return carry

    lax.fori_loop(0, PER_W // 2, pair, 0, unroll=False)
    zout_copy(base + PER_W - 1, 0).wait()
    zout_copy(base + PER_W - 1, 1).wait()

  return lookup


def kernel(token_ids, weight):
  tid_lin = (
      token_ids.T.reshape(TT, 8, BT, 128).transpose(0, 2, 1, 3).reshape(-1)
  ).astype(jnp.int32)
  z = _make_lookup()(tid_lin, weight)
  return z.transpose(2, 4, 0, 1, 3).reshape(16384, 200, DIM)
